# baseline (speedup 1.0000x reference)
import functools

import jax
import jax.numpy as jnp
from jax import lax
from jax.experimental import pallas as pl
from jax.experimental.pallas import tpu as pltpu
from jax.experimental.pallas import tpu_sc as plsc

_TA = 256
_TB = 256


def _router_body(
    x_ref, g_ref, b_ref, w1_ref, b1_ref, w2_ref, b2_ref,
    eid_ref, prank_ref, counts_ref, run_ref,
):
    @pl.when(pl.program_id(0) == 0)
    def _():
        run_ref[...] = jnp.zeros_like(run_ref)

    x = x_ref[...]
    mu = jnp.mean(x, axis=-1, keepdims=True)
    var = jnp.mean((x - mu) ** 2, axis=-1, keepdims=True)
    xn = (x - mu) / jnp.sqrt(var + 1e-5) * g_ref[...] + b_ref[...]
    rh = jnp.maximum(
        jnp.dot(xn, w1_ref[...], preferred_element_type=jnp.float32) + b1_ref[...],
        0.0,
    )
    logits = jnp.dot(rh, w2_ref[...], preferred_element_type=jnp.float32) + b2_ref[...]
    m = jnp.max(logits, axis=-1, keepdims=True)
    p = jnp.exp(logits - m)
    probs = p / jnp.sum(p, axis=-1, keepdims=True)
    e_count = probs.shape[-1]
    is_max = probs == jnp.max(probs, axis=-1, keepdims=True)
    idx = lax.broadcasted_iota(jnp.int32, probs.shape, 1)
    eid = jnp.min(jnp.where(is_max, idx, e_count), axis=-1, keepdims=True)
    eid_ref[...] = eid.astype(jnp.int32)

    oh = (eid == lax.broadcasted_iota(jnp.int32, (_TA, e_count), 1)).astype(
        jnp.float32
    )
    r_i = lax.broadcasted_iota(jnp.int32, (_TA, _TA), 0)
    c_i = lax.broadcasted_iota(jnp.int32, (_TA, _TA), 1)
    lt = (c_i < r_i).astype(jnp.float32)
    within = jnp.dot(lt, oh, preferred_element_type=jnp.float32)
    prank = jnp.sum(oh * (within + run_ref[...]), axis=-1, keepdims=True)
    prank_ref[...] = prank.astype(jnp.int32)
    run_new = run_ref[...] + jnp.sum(oh, axis=0, keepdims=True)
    run_ref[...] = run_new
    counts_ref[...] = run_new.astype(jnp.int32)


def _route(x, ln_gamma, ln_beta, Wr1, br1, Wr2, br2, n_experts):
    n, d = x.shape
    hr = Wr1.shape[1]
    e = Wr2.shape[1]
    grid = (n // _TA,)
    eid2d, prank2d, counts2d = pl.pallas_call(
        _router_body,
        grid=grid,
        in_specs=[
            pl.BlockSpec((_TA, d), lambda i: (i, 0)),
            pl.BlockSpec((1, d), lambda i: (0, 0)),
            pl.BlockSpec((1, d), lambda i: (0, 0)),
            pl.BlockSpec((d, hr), lambda i: (0, 0)),
            pl.BlockSpec((1, hr), lambda i: (0, 0)),
            pl.BlockSpec((hr, e), lambda i: (0, 0)),
            pl.BlockSpec((1, e), lambda i: (0, 0)),
        ],
        out_specs=[
            pl.BlockSpec((_TA, 1), lambda i: (i, 0)),
            pl.BlockSpec((_TA, 1), lambda i: (i, 0)),
            pl.BlockSpec((1, e), lambda i: (0, 0)),
        ],
        out_shape=[
            jax.ShapeDtypeStruct((n, 1), jnp.int32),
            jax.ShapeDtypeStruct((n, 1), jnp.int32),
            jax.ShapeDtypeStruct((1, e), jnp.int32),
        ],
        scratch_shapes=[pltpu.VMEM((1, n_experts), jnp.float32)],
    )(
        x,
        ln_gamma.reshape(1, d),
        ln_beta.reshape(1, d),
        Wr1,
        br1.reshape(1, hr),
        Wr2,
        br2.reshape(1, e),
    )
    return eid2d[:, 0], prank2d[:, 0], counts2d[0]


def _sc_scatter(rows, idx, n_out):
    n, d = rows.shape
    info = plsc.get_sparse_core_info()
    nw = info.num_cores * info.num_subcores
    b_per_w = n // nw
    mesh = plsc.VectorSubcoreMesh(core_axis_name="c", subcore_axis_name="s")

    @functools.partial(
        pl.kernel,
        mesh=mesh,
        out_type=jax.ShapeDtypeStruct((n_out, d), rows.dtype),
        scratch_types=[
            pltpu.VMEM((b_per_w,), jnp.int32),
            pltpu.VMEM((b_per_w, d), rows.dtype),
            pltpu.SemaphoreType.DMA,
        ],
    )
    def k(rows_hbm, idx_hbm, out_hbm, idx_v, rows_v, sem):
        wid = lax.axis_index("s") * info.num_cores + lax.axis_index("c")
        base = wid * b_per_w
        pltpu.sync_copy(idx_hbm.at[pl.ds(base, b_per_w)], idx_v)
        pltpu.sync_copy(rows_hbm.at[pl.ds(base, b_per_w)], rows_v)
        pltpu.async_copy(rows_v, out_hbm.at[idx_v], sem).wait()

    return k(rows, idx)


def _sc_gather(table, idx):
    n, d = table.shape
    info = plsc.get_sparse_core_info()
    nw = info.num_cores * info.num_subcores
    b_per_w = n // nw
    mesh = plsc.VectorSubcoreMesh(core_axis_name="c", subcore_axis_name="s")

    @functools.partial(
        pl.kernel,
        mesh=mesh,
        out_type=jax.ShapeDtypeStruct((n, d), table.dtype),
        scratch_types=[
            pltpu.VMEM((b_per_w,), jnp.int32),
            pltpu.VMEM((b_per_w, d), table.dtype),
            pltpu.SemaphoreType.DMA,
        ],
    )
    def k(table_hbm, idx_hbm, out_hbm, idx_v, rows_v, sem):
        wid = lax.axis_index("s") * info.num_cores + lax.axis_index("c")
        base = wid * b_per_w
        pltpu.sync_copy(idx_hbm.at[pl.ds(base, b_per_w)], idx_v)
        pltpu.async_copy(table_hbm.at[idx_v], rows_v, sem).wait()
        pltpu.sync_copy(rows_v, out_hbm.at[pl.ds(base, b_per_w)])

    return k(table, idx)


def _mlp_body(
    e_ref, t_ref, rs_ref, re_ref, fst_ref,
    xs_ref, g_ref, beta_ref, w1_ref, b1_ref, w2_ref, b2_ref, o_ref,
):
    s = pl.program_id(0)
    rs = rs_ref[s]
    re = re_ref[s]
    fst = fst_ref[s]
    t = t_ref[s]

    @pl.when(rs < re)
    def _():
        x = xs_ref[...]
        mu = jnp.mean(x, axis=-1, keepdims=True)
        var = jnp.mean((x - mu) ** 2, axis=-1, keepdims=True)
        xn = (x - mu) / jnp.sqrt(var + 1e-5) * g_ref[...] + beta_ref[...]
        h = jnp.maximum(
            jnp.dot(xn, w1_ref[0], preferred_element_type=jnp.float32) + b1_ref[0],
            0.0,
        )
        y = jnp.dot(h, w2_ref[0], preferred_element_type=jnp.float32) + b2_ref[0]
        rows = t * _TB + lax.broadcasted_iota(jnp.int32, (_TB, 1), 0)
        contrib = jnp.where((rows >= rs) & (rows < re), y, 0.0)

        @pl.when(fst == 1)
        def _():
            o_ref[...] = x + contrib

        @pl.when(fst == 0)
        def _():
            o_ref[...] = o_ref[...] + contrib


def _grouped_mlp(xs, ln_gamma, ln_beta, W1, b1, W2, b2, e_of, t_of, rs, re, fst):
    n, d = xs.shape
    e, _, h = W1.shape
    n_steps = e_of.shape[0]
    grid_spec = pltpu.PrefetchScalarGridSpec(
        num_scalar_prefetch=5,
        grid=(n_steps,),
        in_specs=[
            pl.BlockSpec((_TB, d), lambda s, ea, ta, ra, rb, fa: (ta[s], 0)),
            pl.BlockSpec((1, d), lambda s, ea, ta, ra, rb, fa: (0, 0)),
            pl.BlockSpec((1, d), lambda s, ea, ta, ra, rb, fa: (0, 0)),
            pl.BlockSpec((1, d, h), lambda s, ea, ta, ra, rb, fa: (ea[s], 0, 0)),
            pl.BlockSpec((1, 1, h), lambda s, ea, ta, ra, rb, fa: (ea[s], 0, 0)),
            pl.BlockSpec((1, h, d), lambda s, ea, ta, ra, rb, fa: (ea[s], 0, 0)),
            pl.BlockSpec((1, 1, d), lambda s, ea, ta, ra, rb, fa: (ea[s], 0, 0)),
        ],
        out_specs=pl.BlockSpec((_TB, d), lambda s, ea, ta, ra, rb, fa: (ta[s], 0)),
    )
    return pl.pallas_call(
        _mlp_body,
        grid_spec=grid_spec,
        out_shape=jax.ShapeDtypeStruct((n, d), jnp.float32),
    )(
        e_of, t_of, rs, re, fst,
        xs,
        ln_gamma.reshape(1, d),
        ln_beta.reshape(1, d),
        W1,
        b1.reshape(e, 1, h),
        W2,
        b2.reshape(e, 1, d),
    )


def _step_metadata(counts, n_rows):
    n_experts = counts.shape[0]
    n_tiles = n_rows // _TB
    n_steps = n_experts + n_tiles
    end = jnp.cumsum(counts)
    start = end - counts
    tiles_e = jnp.where(counts > 0, (end - 1) // _TB - start // _TB + 1, 0)
    step_first = jnp.cumsum(tiles_e) - tiles_e
    total = jnp.sum(tiles_e)
    s = jnp.arange(n_steps, dtype=jnp.int32)
    s_eff = jnp.minimum(s, total - 1)
    e_of = (
        jnp.sum((step_first[None, :] <= s_eff[:, None]).astype(jnp.int32), axis=1) - 1
    ).astype(jnp.int32)
    e_oh = (
        e_of[:, None] == jnp.arange(n_experts, dtype=jnp.int32)[None, :]
    ).astype(jnp.int32)
    sf_g = jnp.sum(e_oh * step_first[None, :], axis=1)
    st_g = jnp.sum(e_oh * start[None, :], axis=1)
    en_g = jnp.sum(e_oh * end[None, :], axis=1)
    t_of = (st_g // _TB + (s_eff - sf_g)).astype(jnp.int32)
    rs = jnp.maximum(st_g, t_of * _TB).astype(jnp.int32)
    re = jnp.minimum(en_g, (t_of + 1) * _TB).astype(jnp.int32)
    valid = s < total
    rs = jnp.where(valid, rs, 1)
    re = jnp.where(valid, re, 0)
    prev_t = jnp.concatenate([jnp.full((1,), -1, jnp.int32), t_of[:-1]])
    fst = (valid & (t_of != prev_t)).astype(jnp.int32)
    return e_of, t_of, rs, re, fst, start


def kernel(x, ln_gamma, ln_beta, Wc, bc, Wr1, br1, Wr2, br2, W1, b1, W2, b2):
    del Wc, bc
    n_experts = W1.shape[0]
    n = x.shape[0]

    eids, prank, counts = _route(
        x, ln_gamma, ln_beta, Wr1, br1, Wr2, br2, n_experts
    )
    e_of, t_of, rs, re, fst, start = _step_metadata(counts, n)

    oh = (
        eids[:, None] == jnp.arange(n_experts, dtype=jnp.int32)[None, :]
    ).astype(jnp.int32)
    dest = (jnp.sum(oh * start[None, :], axis=1) + prank).astype(jnp.int32)

    xs = _sc_scatter(x, dest, n)
    ys = _grouped_mlp(xs, ln_gamma, ln_beta, W1, b1, W2, b2, e_of, t_of, rs, re, fst)
    return _sc_gather(ys, dest)


# speedup vs baseline: 1.0078x; 1.0078x over previous
import functools

import jax
import jax.numpy as jnp
from jax import lax
from jax.experimental import pallas as pl
from jax.experimental.pallas import tpu as pltpu
from jax.experimental.pallas import tpu_sc as plsc

_TA = 256
_TB = 256


def _meta_math(counts_f, n_steps):
    e_count = counts_f.shape[-1]
    r64 = lax.broadcasted_iota(jnp.int32, (e_count, e_count), 0)
    c64 = lax.broadcasted_iota(jnp.int32, (e_count, e_count), 1)
    ut_incl = (r64 <= c64).astype(jnp.float32)
    ut_strict = (r64 < c64).astype(jnp.float32)
    end_i = jnp.dot(counts_f, ut_incl, preferred_element_type=jnp.float32).astype(
        jnp.int32
    )
    counts_i = counts_f.astype(jnp.int32)
    start_i = end_i - counts_i
    tiles_e = jnp.where(
        counts_i > 0, (end_i - 1) // _TB - start_i // _TB + 1, 0
    )
    sfirst_i = jnp.dot(
        tiles_e.astype(jnp.float32), ut_strict, preferred_element_type=jnp.float32
    ).astype(jnp.int32)
    total = jnp.sum(tiles_e, axis=1, keepdims=True)
    s_col = lax.broadcasted_iota(jnp.int32, (n_steps, e_count), 0)
    e_row = lax.broadcasted_iota(jnp.int32, (n_steps, e_count), 1)
    s_eff = jnp.minimum(s_col, total - 1)
    e_of = jnp.sum((sfirst_i <= s_eff).astype(jnp.int32), axis=1, keepdims=True) - 1
    e_oh = (e_of == e_row).astype(jnp.int32)
    sf_g = jnp.sum(e_oh * sfirst_i, axis=1, keepdims=True)
    st_g = jnp.sum(e_oh * start_i, axis=1, keepdims=True)
    en_g = jnp.sum(e_oh * end_i, axis=1, keepdims=True)
    s1 = lax.broadcasted_iota(jnp.int32, (n_steps, 1), 0)
    s1_eff = jnp.minimum(s1, total - 1)
    t_of = st_g // _TB + (s1_eff - sf_g)
    rs = jnp.maximum(st_g, t_of * _TB)
    re = jnp.minimum(en_g, (t_of + 1) * _TB)
    valid = s1 < total
    rs = jnp.where(valid, rs, 1)
    re = jnp.where(valid, re, 0)
    prev_t = jnp.concatenate(
        [jnp.full((1, 1), -1, jnp.int32), t_of[:-1, :]], axis=0
    )
    fst = (valid & (t_of != prev_t)).astype(jnp.int32)
    return e_of, t_of, rs, re, fst, start_i


def _router_body(
    x_ref, g_ref, b_ref, w1_ref, b1_ref, w2_ref, b2_ref,
    eid_ref, prank_ref, start_ref, me_ref, mt_ref, mrs_ref, mre_ref, mf_ref,
    run_ref, *, n_steps,
):
    @pl.when(pl.program_id(0) == 0)
    def _():
        run_ref[...] = jnp.zeros_like(run_ref)

    x = x_ref[...]
    mu = jnp.mean(x, axis=-1, keepdims=True)
    var = jnp.mean((x - mu) ** 2, axis=-1, keepdims=True)
    xn = (x - mu) / jnp.sqrt(var + 1e-5) * g_ref[...] + b_ref[...]
    rh = jnp.maximum(
        jnp.dot(xn, w1_ref[...], preferred_element_type=jnp.float32) + b1_ref[...],
        0.0,
    )
    logits = jnp.dot(rh, w2_ref[...], preferred_element_type=jnp.float32) + b2_ref[...]
    m = jnp.max(logits, axis=-1, keepdims=True)
    p = jnp.exp(logits - m)
    probs = p / jnp.sum(p, axis=-1, keepdims=True)
    e_count = probs.shape[-1]
    is_max = probs == jnp.max(probs, axis=-1, keepdims=True)
    idx = lax.broadcasted_iota(jnp.int32, probs.shape, 1)
    eid = jnp.min(jnp.where(is_max, idx, e_count), axis=-1, keepdims=True)
    eid_ref[...] = eid.astype(jnp.int32)

    oh = (eid == lax.broadcasted_iota(jnp.int32, (_TA, e_count), 1)).astype(
        jnp.float32
    )
    r_i = lax.broadcasted_iota(jnp.int32, (_TA, _TA), 0)
    c_i = lax.broadcasted_iota(jnp.int32, (_TA, _TA), 1)
    lt = (c_i < r_i).astype(jnp.float32)
    within = jnp.dot(lt, oh, preferred_element_type=jnp.float32)
    prank = jnp.sum(oh * (within + run_ref[...]), axis=-1, keepdims=True)
    prank_ref[...] = prank.astype(jnp.int32)
    run_new = run_ref[...] + jnp.sum(oh, axis=0, keepdims=True)
    run_ref[...] = run_new

    @pl.when(pl.program_id(0) == pl.num_programs(0) - 1)
    def _():
        e_of, t_of, m_rs, m_re, m_fst, start_i = _meta_math(run_new, n_steps)
        start_ref[...] = start_i
        me_ref[...] = e_of
        mt_ref[...] = t_of
        mrs_ref[...] = m_rs
        mre_ref[...] = m_re
        mf_ref[...] = m_fst


def _route(x, ln_gamma, ln_beta, Wr1, br1, Wr2, br2, n_experts, n_steps):
    n, d = x.shape
    hr = Wr1.shape[1]
    e = Wr2.shape[1]
    grid = (n // _TA,)
    outs = pl.pallas_call(
        functools.partial(_router_body, n_steps=n_steps),
        grid=grid,
        in_specs=[
            pl.BlockSpec((_TA, d), lambda i: (i, 0)),
            pl.BlockSpec((1, d), lambda i: (0, 0)),
            pl.BlockSpec((1, d), lambda i: (0, 0)),
            pl.BlockSpec((d, hr), lambda i: (0, 0)),
            pl.BlockSpec((1, hr), lambda i: (0, 0)),
            pl.BlockSpec((hr, e), lambda i: (0, 0)),
            pl.BlockSpec((1, e), lambda i: (0, 0)),
        ],
        out_specs=[
            pl.BlockSpec((_TA, 1), lambda i: (i, 0)),
            pl.BlockSpec((_TA, 1), lambda i: (i, 0)),
            pl.BlockSpec((1, e), lambda i: (0, 0)),
        ]
        + [pl.BlockSpec((n_steps, 1), lambda i: (0, 0))] * 5,
        out_shape=[
            jax.ShapeDtypeStruct((n, 1), jnp.int32),
            jax.ShapeDtypeStruct((n, 1), jnp.int32),
            jax.ShapeDtypeStruct((1, e), jnp.int32),
        ]
        + [jax.ShapeDtypeStruct((n_steps, 1), jnp.int32)] * 5,
        scratch_shapes=[pltpu.VMEM((1, n_experts), jnp.float32)],
    )(
        x,
        ln_gamma.reshape(1, d),
        ln_beta.reshape(1, d),
        Wr1,
        br1.reshape(1, hr),
        Wr2,
        br2.reshape(1, e),
    )
    eid2d, prank2d, start2d, me, mt, mrs, mre, mf = outs
    return (
        eid2d[:, 0], prank2d[:, 0], start2d,
        me[:, 0], mt[:, 0], mrs[:, 0], mre[:, 0], mf[:, 0],
    )


def _sc_scatter(rows, idx, n_out):
    n, d = rows.shape
    info = plsc.get_sparse_core_info()
    nw = info.num_cores * info.num_subcores
    b_per_w = n // nw
    mesh = plsc.VectorSubcoreMesh(core_axis_name="c", subcore_axis_name="s")

    @functools.partial(
        pl.kernel,
        mesh=mesh,
        out_type=jax.ShapeDtypeStruct((n_out, d), rows.dtype),
        scratch_types=[
            pltpu.VMEM((b_per_w,), jnp.int32),
            pltpu.VMEM((b_per_w, d), rows.dtype),
            pltpu.SemaphoreType.DMA,
        ],
    )
    def k(rows_hbm, idx_hbm, out_hbm, idx_v, rows_v, sem):
        wid = lax.axis_index("s") * info.num_cores + lax.axis_index("c")
        base = wid * b_per_w
        pltpu.sync_copy(idx_hbm.at[pl.ds(base, b_per_w)], idx_v)
        pltpu.sync_copy(rows_hbm.at[pl.ds(base, b_per_w)], rows_v)
        pltpu.async_copy(rows_v, out_hbm.at[idx_v], sem).wait()

    return k(rows, idx)


def _sc_gather(table, idx):
    n, d = table.shape
    info = plsc.get_sparse_core_info()
    nw = info.num_cores * info.num_subcores
    b_per_w = n // nw
    mesh = plsc.VectorSubcoreMesh(core_axis_name="c", subcore_axis_name="s")

    @functools.partial(
        pl.kernel,
        mesh=mesh,
        out_type=jax.ShapeDtypeStruct((n, d), table.dtype),
        scratch_types=[
            pltpu.VMEM((b_per_w,), jnp.int32),
            pltpu.VMEM((b_per_w, d), table.dtype),
            pltpu.SemaphoreType.DMA,
        ],
    )
    def k(table_hbm, idx_hbm, out_hbm, idx_v, rows_v, sem):
        wid = lax.axis_index("s") * info.num_cores + lax.axis_index("c")
        base = wid * b_per_w
        pltpu.sync_copy(idx_hbm.at[pl.ds(base, b_per_w)], idx_v)
        pltpu.async_copy(table_hbm.at[idx_v], rows_v, sem).wait()
        pltpu.sync_copy(rows_v, out_hbm.at[pl.ds(base, b_per_w)])

    return k(table, idx)


def _mlp_body(
    e_ref, t_ref, rs_ref, re_ref, fst_ref,
    xs_ref, g_ref, beta_ref, w1_ref, b1_ref, w2_ref, b2_ref, o_ref,
):
    s = pl.program_id(0)
    rs = rs_ref[s]
    re = re_ref[s]
    fst = fst_ref[s]
    t = t_ref[s]

    @pl.when(rs < re)
    def _():
        x = xs_ref[...]
        mu = jnp.mean(x, axis=-1, keepdims=True)
        var = jnp.mean((x - mu) ** 2, axis=-1, keepdims=True)
        xn = (x - mu) / jnp.sqrt(var + 1e-5) * g_ref[...] + beta_ref[...]
        h = jnp.maximum(
            jnp.dot(xn, w1_ref[0], preferred_element_type=jnp.float32) + b1_ref[0],
            0.0,
        )
        y = jnp.dot(h, w2_ref[0], preferred_element_type=jnp.float32) + b2_ref[0]
        rows = t * _TB + lax.broadcasted_iota(jnp.int32, (_TB, 1), 0)
        contrib = jnp.where((rows >= rs) & (rows < re), y, 0.0)

        @pl.when(fst == 1)
        def _():
            o_ref[...] = x + contrib

        @pl.when(fst == 0)
        def _():
            o_ref[...] = o_ref[...] + contrib


def _grouped_mlp(xs, ln_gamma, ln_beta, W1, b1, W2, b2, e_of, t_of, rs, re, fst):
    n, d = xs.shape
    e, _, h = W1.shape
    n_steps = e_of.shape[0]
    grid_spec = pltpu.PrefetchScalarGridSpec(
        num_scalar_prefetch=5,
        grid=(n_steps,),
        in_specs=[
            pl.BlockSpec((_TB, d), lambda s, ea, ta, ra, rb, fa: (ta[s], 0)),
            pl.BlockSpec((1, d), lambda s, ea, ta, ra, rb, fa: (0, 0)),
            pl.BlockSpec((1, d), lambda s, ea, ta, ra, rb, fa: (0, 0)),
            pl.BlockSpec((1, d, h), lambda s, ea, ta, ra, rb, fa: (ea[s], 0, 0)),
            pl.BlockSpec((1, 1, h), lambda s, ea, ta, ra, rb, fa: (ea[s], 0, 0)),
            pl.BlockSpec((1, h, d), lambda s, ea, ta, ra, rb, fa: (ea[s], 0, 0)),
            pl.BlockSpec((1, 1, d), lambda s, ea, ta, ra, rb, fa: (ea[s], 0, 0)),
        ],
        out_specs=pl.BlockSpec((_TB, d), lambda s, ea, ta, ra, rb, fa: (ta[s], 0)),
    )
    return pl.pallas_call(
        _mlp_body,
        grid_spec=grid_spec,
        out_shape=jax.ShapeDtypeStruct((n, d), jnp.float32),
    )(
        e_of, t_of, rs, re, fst,
        xs,
        ln_gamma.reshape(1, d),
        ln_beta.reshape(1, d),
        W1,
        b1.reshape(e, 1, h),
        W2,
        b2.reshape(e, 1, d),
    )


def kernel(x, ln_gamma, ln_beta, Wc, bc, Wr1, br1, Wr2, br2, W1, b1, W2, b2):
    del Wc, bc
    n_experts = W1.shape[0]
    n = x.shape[0]
    n_steps = n_experts + n // _TB

    eids, prank, start2d, e_of, t_of, rs, re, fst = _route(
        x, ln_gamma, ln_beta, Wr1, br1, Wr2, br2, n_experts, n_steps
    )

    oh = (
        eids[:, None] == jnp.arange(n_experts, dtype=jnp.int32)[None, :]
    ).astype(jnp.int32)
    dest = (jnp.sum(oh * start2d[0][None, :], axis=1) + prank).astype(jnp.int32)

    xs = _sc_scatter(x, dest, n)
    ys = _grouped_mlp(xs, ln_gamma, ln_beta, W1, b1, W2, b2, e_of, t_of, rs, re, fst)
    return _sc_gather(ys, dest)
